# baseline (speedup 1.0000x reference)
import functools

import jax
import jax.numpy as jnp
from jax.experimental import pallas as pl
from jax.experimental.pallas import tpu as pltpu


def _mm_body(x_ref, w_ref, o_ref, *, k_blk):
    k = pl.program_id(1)
    x_blk = x_ref[:, pl.ds(k * k_blk, k_blk)]
    w_blk = w_ref[...].astype(jnp.bfloat16)
    acc = jax.lax.dot_general(
        x_blk, w_blk,
        dimension_numbers=(((1,), (1,)), ((), ())),
        preferred_element_type=jnp.float32)

    @pl.when(k == 0)
    def _():
        o_ref[...] = acc

    @pl.when(k > 0)
    def _():
        o_ref[...] += acc


@functools.partial(jax.jit, static_argnames=("n_blk", "k_blk"))
def _spmm(x, W, n_blk=512, k_blk=2048):
    m, kdim = x.shape
    ndim = W.shape[0]
    x16 = x.astype(jnp.bfloat16)
    grid = (ndim // n_blk, kdim // k_blk)
    return pl.pallas_call(
        functools.partial(_mm_body, k_blk=k_blk),
        grid=grid,
        in_specs=[
            pl.BlockSpec((m, kdim), lambda n, k: (0, 0)),
            pl.BlockSpec((n_blk, k_blk), lambda n, k: (n, k)),
        ],
        out_specs=pl.BlockSpec((m, n_blk), lambda n, k: (0, n)),
        out_shape=jax.ShapeDtypeStruct((m, ndim), jnp.float32),
        compiler_params=pltpu.CompilerParams(
            dimension_semantics=("arbitrary", "arbitrary")),
    )(x16, W)


def kernel(x, W, bias):
    del bias
    return _spmm(x, W)


# speedup vs baseline: 1.2362x; 1.2362x over previous
import functools

import jax
import jax.numpy as jnp
from jax.experimental import pallas as pl
from jax.experimental.pallas import tpu as pltpu


def _mm_body(x_ref, w_ref, o_ref):
    w_blk = w_ref[...].astype(jnp.bfloat16)
    o_ref[...] = jax.lax.dot_general(
        x_ref[...], w_blk,
        dimension_numbers=(((1,), (1,)), ((), ())),
        preferred_element_type=jnp.float32)


@functools.partial(jax.jit, static_argnames=("n_blk",))
def _spmm(x, W, n_blk=256):
    m, kdim = x.shape
    ndim = W.shape[0]
    x16 = x.astype(jnp.bfloat16)
    return pl.pallas_call(
        _mm_body,
        grid=(ndim // n_blk,),
        in_specs=[
            pl.BlockSpec((m, kdim), lambda n: (0, 0)),
            pl.BlockSpec((n_blk, kdim), lambda n: (n, 0)),
        ],
        out_specs=pl.BlockSpec((m, n_blk), lambda n: (0, n)),
        out_shape=jax.ShapeDtypeStruct((m, ndim), jnp.float32),
        compiler_params=pltpu.CompilerParams(
            dimension_semantics=("parallel",)),
    )(x16, W)


def kernel(x, W, bias):
    del bias
    return _spmm(x, W)


# speedup vs baseline: 1.2863x; 1.0405x over previous
import functools

import jax
import jax.numpy as jnp
from jax.experimental import pallas as pl
from jax.experimental.pallas import tpu as pltpu


def _mm_body(x_ref, w_ref, o_ref, x16_ref):
    @pl.when(pl.program_id(0) == 0)
    def _():
        x16_ref[...] = x_ref[...].astype(jnp.bfloat16)

    w_blk = w_ref[...].astype(jnp.bfloat16)
    o_ref[...] = jax.lax.dot_general(
        x16_ref[...], w_blk,
        dimension_numbers=(((1,), (1,)), ((), ())),
        preferred_element_type=jnp.float32)


@functools.partial(jax.jit, static_argnames=("n_blk",))
def _spmm(x, W, n_blk=256):
    m, kdim = x.shape
    ndim = W.shape[0]
    return pl.pallas_call(
        _mm_body,
        grid=(ndim // n_blk,),
        in_specs=[
            pl.BlockSpec((m, kdim), lambda n: (0, 0)),
            pl.BlockSpec((n_blk, kdim), lambda n: (n, 0)),
        ],
        out_specs=pl.BlockSpec((m, n_blk), lambda n: (0, n)),
        out_shape=jax.ShapeDtypeStruct((m, ndim), jnp.float32),
        scratch_shapes=[pltpu.VMEM((m, kdim), jnp.bfloat16)],
        compiler_params=pltpu.CompilerParams(
            dimension_semantics=("arbitrary",)),
    )(x, W)


def kernel(x, W, bias):
    del bias
    return _spmm(x, W)
